# baseline (speedup 1.0000x reference)
import functools

import jax
import jax.numpy as jnp
from jax import lax
from jax.experimental import pallas as pl
from jax.experimental.pallas import tpu as pltpu
from jax.experimental.pallas import tpu_sc as plsc

N_TERMS = 1000000
TERM_DIM = 64
REL_DIM = 8
N_RELS = 40
B = 16384

_PAIR = 2 * TERM_DIM
_NPACK = N_TERMS // 2

_NC = 2
_NS = 16
_NW = _NC * _NS
_BPW = B // _NW
_CHUNK = 128
_NCHUNK = _BPW // _CHUNK

_BLK = 4096
_NB = B // _BLK

_RL = 32768
_RH = _RL // 2
_RG = -(-N_TERMS // _RL)
_NPACK2 = _RG * _RH


def _repack_body(tt_ref, out_ref):
    xt = jnp.swapaxes(tt_ref[...], 0, 1)
    out_ref[...] = jnp.concatenate([xt[:_RH, :], xt[_RH:, :]], axis=1)


_repack = pl.pallas_call(
    _repack_body,
    grid=(_RG,),
    in_specs=[pl.BlockSpec((TERM_DIM, _RL), lambda i: (0, i))],
    out_specs=pl.BlockSpec((_RH, _PAIR), lambda i: (i, 0)),
    out_shape=jax.ShapeDtypeStruct((_NPACK2, _PAIR), jnp.float32),
)


def _sc_gather_body(tl_ref, tr_ref, tab_ref, out_l_ref, out_r_ref,
                    idx_v, rows, sem):
    wid = lax.axis_index("s") * _NC + lax.axis_index("c")
    row0 = wid * _NCHUNK
    base = wid * _BPW
    for src, dst in ((tl_ref, out_l_ref), (tr_ref, out_r_ref)):
        pltpu.sync_copy(src.at[pl.ds(row0, _NCHUNK)], idx_v)
        copies = []
        for j in range(_NCHUNK):
            copies.append(pltpu.async_copy(
                tab_ref.at[idx_v.at[j]],
                rows.at[pl.ds(j * _CHUNK, _CHUNK)], sem))
        for c in copies:
            c.wait()
        pltpu.sync_copy(rows, dst.at[pl.ds(base, _BPW)])


_sc_gather = functools.partial(
    pl.kernel,
    mesh=plsc.VectorSubcoreMesh(core_axis_name="c", subcore_axis_name="s"),
    out_type=[
        jax.ShapeDtypeStruct((B, _PAIR), jnp.float32),
        jax.ShapeDtypeStruct((B, _PAIR), jnp.float32),
    ],
    scratch_types=[
        pltpu.VMEM((_NCHUNK, _CHUNK), jnp.int32),
        pltpu.VMEM((_BPW, _PAIR), jnp.float32),
        pltpu.SemaphoreType.DMA,
    ],
    compiler_params=pltpu.CompilerParams(use_tc_tiling_on_sc=True),
)(_sc_gather_body)


def _tc_body(rels_ref, gl_ref, gr_ref, pl_ref, pr_ref, relv_ref, wf_ref,
             b_ref, tm_ref, to_ref, energy_ref, ninter_ref, nrel_ref):
    gl = gl_ref[...]
    gr = gr_ref[...]
    x = jnp.where(pl_ref[...] > 0.5, gl[:, TERM_DIM:], gl[:, :TERM_DIM])
    y = jnp.where(pr_ref[...] > 0.5, gr[:, TERM_DIM:], gr[:, :TERM_DIM])
    t = jnp.dot(x, wf_ref[...], preferred_element_type=jnp.float32)
    y8 = jnp.concatenate([y] * REL_DIM, axis=1)
    p = t * y8
    sel = (lax.broadcasted_iota(jnp.int32, (REL_DIM * TERM_DIM, REL_DIM), 0)
           // TERM_DIM
           == lax.broadcasted_iota(jnp.int32, (REL_DIM * TERM_DIM, REL_DIM), 1)
           ).astype(jnp.float32)
    inter = jnp.dot(p, sel, preferred_element_type=jnp.float32) + b_ref[...]
    oh = (rels_ref[...] ==
          lax.broadcasted_iota(jnp.int32, (_BLK, N_RELS), 1).astype(jnp.float32)
          ).astype(jnp.float32)
    r = jnp.dot(oh, relv_ref[...], preferred_element_type=jnp.float32,
                precision=lax.Precision.HIGHEST)
    energy = jnp.sum(inter * r, axis=1, keepdims=True)
    energy_ref[...] = energy * tm_ref[...] + to_ref[...]
    ninter_ref[...] = jnp.sum(inter * inter, axis=1, keepdims=True)
    nrel_ref[...] = jnp.sum(r * r, axis=1, keepdims=True)


_tc_call = pl.pallas_call(
    _tc_body,
    grid=(_NB,),
    in_specs=[
        pl.BlockSpec((_BLK, 1), lambda i: (i, 0)),
        pl.BlockSpec((_BLK, _PAIR), lambda i: (i, 0)),
        pl.BlockSpec((_BLK, _PAIR), lambda i: (i, 0)),
        pl.BlockSpec((_BLK, 1), lambda i: (i, 0)),
        pl.BlockSpec((_BLK, 1), lambda i: (i, 0)),
        pl.BlockSpec((N_RELS, REL_DIM), lambda i: (0, 0)),
        pl.BlockSpec((TERM_DIM, REL_DIM * TERM_DIM), lambda i: (0, 0)),
        pl.BlockSpec((1, REL_DIM), lambda i: (0, 0)),
        pl.BlockSpec((1, 1), lambda i: (0, 0)),
        pl.BlockSpec((1, 1), lambda i: (0, 0)),
    ],
    out_specs=[
        pl.BlockSpec((_BLK, 1), lambda i: (i, 0)),
        pl.BlockSpec((_BLK, 1), lambda i: (i, 0)),
        pl.BlockSpec((_BLK, 1), lambda i: (i, 0)),
    ],
    out_shape=[
        jax.ShapeDtypeStruct((B, 1), jnp.float32),
        jax.ShapeDtypeStruct((B, 1), jnp.float32),
        jax.ShapeDtypeStruct((B, 1), jnp.float32),
    ],
)


def kernel(rels, terms_L, terms_R, term_vecs, rel_vecs, W, b,
           truth_multiplier, truth_offset):
    ttab = jnp.swapaxes(term_vecs, 0, 1)
    tab = _repack(ttab)
    tl = terms_L.astype(jnp.int32)
    tr = terms_R.astype(jnp.int32)
    off_l = tl % _RL
    off_r = tr % _RL
    tl_phys = ((tl // _RL) * _RH + off_l % _RH).reshape(_NW * _NCHUNK, _CHUNK)
    tr_phys = ((tr // _RL) * _RH + off_r % _RH).reshape(_NW * _NCHUNK, _CHUNK)
    g_l, g_r = _sc_gather(tl_phys, tr_phys, tab)
    par_l = (off_l >= _RH).astype(jnp.float32).reshape(B, 1)
    par_r = (off_r >= _RH).astype(jnp.float32).reshape(B, 1)
    wf = jnp.transpose(W, (1, 0, 2)).reshape(TERM_DIM, REL_DIM * TERM_DIM)
    relsf = rels.astype(jnp.float32).reshape(B, 1)
    b_row = b.reshape(1, REL_DIM)
    tm = truth_multiplier.reshape(1, 1)
    to = truth_offset.reshape(1, 1)
    energy, ninter, nrel = _tc_call(relsf, g_l, g_r, par_l, par_r, rel_vecs,
                                    wf, b_row, tm, to)
    return energy.reshape(B), ninter.reshape(B), nrel.reshape(B)


# speedup vs baseline: 1.0090x; 1.0090x over previous
import functools

import jax
import jax.numpy as jnp
from jax import lax
from jax.experimental import pallas as pl
from jax.experimental.pallas import tpu as pltpu
from jax.experimental.pallas import tpu_sc as plsc

N_TERMS = 1000000
TERM_DIM = 64
REL_DIM = 8
N_RELS = 40
B = 16384

_PAIR = 2 * TERM_DIM
_NPACK = N_TERMS // 2

_NC = 2
_NS = 16
_NW = _NC * _NS
_BPW = B // _NW
_CHUNK = 128
_NCHUNK = _BPW // _CHUNK

_BLK = 2048
_NB = B // _BLK

_RL = 32768
_RH = _RL // 2
_RG = -(-N_TERMS // _RL)
_NPACK2 = _RG * _RH


def _repack_body(tt_ref, out_ref):
    xt = jnp.swapaxes(tt_ref[...], 0, 1)
    out_ref[...] = jnp.concatenate([xt[:_RH, :], xt[_RH:, :]], axis=1)


_repack = pl.pallas_call(
    _repack_body,
    grid=(_RG,),
    in_specs=[pl.BlockSpec((TERM_DIM, _RL), lambda i: (0, i))],
    out_specs=pl.BlockSpec((_RH, _PAIR), lambda i: (i, 0)),
    out_shape=jax.ShapeDtypeStruct((_NPACK2, _PAIR), jnp.float32),
)


def _sc_gather_body(tl_ref, tr_ref, tab_ref, out_l_ref, out_r_ref,
                    idx_v, rows, sem):
    wid = lax.axis_index("s") * _NC + lax.axis_index("c")
    base = wid * _BPW
    for src, dst in ((tl_ref, out_l_ref), (tr_ref, out_r_ref)):
        pltpu.sync_copy(src.at[pl.ds(base, _BPW)], idx_v)
        copies = []
        for j in range(_NCHUNK):
            copies.append(pltpu.async_copy(
                tab_ref.at[idx_v.at[pl.ds(j * _CHUNK, _CHUNK)]],
                rows.at[pl.ds(j * _CHUNK, _CHUNK)], sem))
        for c in copies:
            c.wait()
        pltpu.sync_copy(rows, dst.at[pl.ds(base, _BPW)])


_sc_gather = functools.partial(
    pl.kernel,
    mesh=plsc.VectorSubcoreMesh(core_axis_name="c", subcore_axis_name="s"),
    out_type=[
        jax.ShapeDtypeStruct((B, _PAIR), jnp.float32),
        jax.ShapeDtypeStruct((B, _PAIR), jnp.float32),
    ],
    scratch_types=[
        pltpu.VMEM((_BPW,), jnp.int32),
        pltpu.VMEM((_BPW, _PAIR), jnp.float32),
        pltpu.SemaphoreType.DMA,
    ],
    compiler_params=pltpu.CompilerParams(use_tc_tiling_on_sc=True),
)(_sc_gather_body)


def _tc_body(rels_ref, gl_ref, gr_ref, pl_ref, pr_ref, relv_ref, wf_ref,
             b_ref, tm_ref, to_ref, energy_ref, ninter_ref, nrel_ref):
    gl = gl_ref[...]
    gr = gr_ref[...]
    x = jnp.where(pl_ref[...] > 0.5, gl[:, TERM_DIM:], gl[:, :TERM_DIM])
    y = jnp.where(pr_ref[...] > 0.5, gr[:, TERM_DIM:], gr[:, :TERM_DIM])
    t = jnp.dot(x, wf_ref[...], preferred_element_type=jnp.float32)
    y8 = jnp.concatenate([y] * REL_DIM, axis=1)
    p = t * y8
    sel = (lax.broadcasted_iota(jnp.int32, (REL_DIM * TERM_DIM, REL_DIM), 0)
           // TERM_DIM
           == lax.broadcasted_iota(jnp.int32, (REL_DIM * TERM_DIM, REL_DIM), 1)
           ).astype(jnp.float32)
    inter = jnp.dot(p, sel, preferred_element_type=jnp.float32) + b_ref[...]
    oh = (rels_ref[...] ==
          lax.broadcasted_iota(jnp.int32, (_BLK, N_RELS), 1).astype(jnp.float32)
          ).astype(jnp.float32)
    r = jnp.dot(oh, relv_ref[...], preferred_element_type=jnp.float32,
                precision=lax.Precision.HIGHEST)
    energy = jnp.sum(inter * r, axis=1, keepdims=True)
    energy_ref[...] = energy * tm_ref[...] + to_ref[...]
    ninter_ref[...] = jnp.sum(inter * inter, axis=1, keepdims=True)
    nrel_ref[...] = jnp.sum(r * r, axis=1, keepdims=True)


_tc_call = pl.pallas_call(
    _tc_body,
    grid=(_NB,),
    in_specs=[
        pl.BlockSpec((_BLK, 1), lambda i: (i, 0)),
        pl.BlockSpec((_BLK, _PAIR), lambda i: (i, 0)),
        pl.BlockSpec((_BLK, _PAIR), lambda i: (i, 0)),
        pl.BlockSpec((_BLK, 1), lambda i: (i, 0)),
        pl.BlockSpec((_BLK, 1), lambda i: (i, 0)),
        pl.BlockSpec((N_RELS, REL_DIM), lambda i: (0, 0)),
        pl.BlockSpec((TERM_DIM, REL_DIM * TERM_DIM), lambda i: (0, 0)),
        pl.BlockSpec((1, REL_DIM), lambda i: (0, 0)),
        pl.BlockSpec((1, 1), lambda i: (0, 0)),
        pl.BlockSpec((1, 1), lambda i: (0, 0)),
    ],
    out_specs=[
        pl.BlockSpec((_BLK, 1), lambda i: (i, 0)),
        pl.BlockSpec((_BLK, 1), lambda i: (i, 0)),
        pl.BlockSpec((_BLK, 1), lambda i: (i, 0)),
    ],
    out_shape=[
        jax.ShapeDtypeStruct((B, 1), jnp.float32),
        jax.ShapeDtypeStruct((B, 1), jnp.float32),
        jax.ShapeDtypeStruct((B, 1), jnp.float32),
    ],
)


def kernel(rels, terms_L, terms_R, term_vecs, rel_vecs, W, b,
           truth_multiplier, truth_offset):
    ttab = jnp.swapaxes(term_vecs, 0, 1)
    tab = _repack(ttab)
    tl = terms_L.astype(jnp.int32)
    tr = terms_R.astype(jnp.int32)
    off_l = tl % _RL
    off_r = tr % _RL
    tl_phys = (tl // _RL) * _RH + off_l % _RH
    tr_phys = (tr // _RL) * _RH + off_r % _RH
    g_l, g_r = _sc_gather(tl_phys, tr_phys, tab)
    par_l = (off_l >= _RH).astype(jnp.float32).reshape(B, 1)
    par_r = (off_r >= _RH).astype(jnp.float32).reshape(B, 1)
    wf = jnp.transpose(W, (1, 0, 2)).reshape(TERM_DIM, REL_DIM * TERM_DIM)
    relsf = rels.astype(jnp.float32).reshape(B, 1)
    b_row = b.reshape(1, REL_DIM)
    tm = truth_multiplier.reshape(1, 1)
    to = truth_offset.reshape(1, 1)
    energy, ninter, nrel = _tc_call(relsf, g_l, g_r, par_l, par_r, rel_vecs,
                                    wf, b_row, tm, to)
    return energy.reshape(B), ninter.reshape(B), nrel.reshape(B)
